# baseline (speedup 1.0000x reference)
import functools

import jax
import jax.numpy as jnp
from jax import lax
from jax.experimental import pallas as pl
from jax.experimental.pallas import tpu as pltpu
from jax.experimental.pallas import tpu_sc as plsc

NUM_ROWS = 100000
EMBED_DIM = 64
BATCH = 16384

NC = 2
NS = 16
L = 16
NW = NC * NS
B_PER_W = BATCH // NW
CHUNK = 256
N_CHUNKS = B_PER_W // CHUNK
GROUPS = CHUNK // L
CW = 2048


SPLIT = 51200
PACK_BLOCKS = SPLIT // CW


def _pack_body(lo_ref, hi_ref, dst_ref):
    lo = jnp.transpose(lo_ref[...], (1, 0))
    hi = jnp.transpose(hi_ref[...], (1, 0))
    dst_ref[...] = jnp.concatenate([lo, hi], axis=1)


def _pack_tables(tab_t):
    return pl.pallas_call(
        _pack_body,
        grid=(PACK_BLOCKS,),
        in_specs=[
            pl.BlockSpec((EMBED_DIM, CW), lambda c: (0, c)),
            pl.BlockSpec((EMBED_DIM, CW),
                         lambda c: (0, jnp.minimum(c + PACK_BLOCKS,
                                                   2 * PACK_BLOCKS - 2))),
        ],
        out_specs=pl.BlockSpec((CW, 2 * EMBED_DIM), lambda c: (c, 0)),
        out_shape=jax.ShapeDtypeStruct((SPLIT, 2 * EMBED_DIM), jnp.float32),
    )(tab_t, tab_t)


def _perm_xor(v, s, lanes):
    idx = jnp.bitwise_xor(lanes, s)
    return jnp.take_along_axis(v, idx, axis=0, mode="promise_in_bounds")


def _butterfly_rowsum(vecs, lanes):
    s = 1
    while len(vecs) > 1:
        mask = (jnp.bitwise_and(lanes, s) == 0)
        nxt = []
        for i in range(0, len(vecs), 2):
            a, b = vecs[i], vecs[i + 1]
            nxt.append(jnp.where(mask, a + _perm_xor(a, s, lanes),
                                 b + _perm_xor(b, s, lanes)))
        vecs = nxt
        s *= 2
    return vecs[0]


def _tt_body(uid_hbm, iid_hbm, utab_hbm, itab_hbm, out_hbm,
             uid_v, iid_v, urows_v, irows_v, out_v, sem_u, sem_i):
    wid = lax.axis_index("s") * NC + lax.axis_index("c")
    base = wid * B_PER_W

    pltpu.sync_copy(uid_hbm.at[pl.ds(base, B_PER_W)], uid_v)
    pltpu.sync_copy(iid_hbm.at[pl.ds(base, B_PER_W)], iid_v)

    lanes = lax.iota(jnp.int32, L)

    def block(b, carry):
        uvec = uid_v[pl.ds(b * L, L)]
        ivec = iid_v[pl.ds(b * L, L)]
        upair = jnp.where(uvec >= SPLIT, uvec - SPLIT, uvec)
        ipair = jnp.where(ivec >= SPLIT, ivec - SPLIT, ivec)
        for j in range(L):
            pltpu.async_copy(utab_hbm.at[upair[j]],
                             urows_v.at[j], sem_u).wait()
            pltpu.async_copy(itab_hbm.at[ipair[j]],
                             irows_v.at[j], sem_i).wait()
        partials = []
        for j in range(L):
            uge = uvec[j] >= SPLIT
            ige = ivec[j] >= SPLIT
            p = None
            for k in range(EMBED_DIM // L):
                u_lo = urows_v[j, pl.ds(k * L, L)]
                u_hi = urows_v[j, pl.ds(EMBED_DIM + k * L, L)]
                v_lo = irows_v[j, pl.ds(k * L, L)]
                v_hi = irows_v[j, pl.ds(EMBED_DIM + k * L, L)]
                u = jnp.where(uge, u_hi, u_lo)
                v = jnp.where(ige, v_hi, v_lo)
                p = u * v if p is None else p + u * v
            partials.append(p)
        score = _butterfly_rowsum(partials, lanes)
        prob = 1.0 / (1.0 + jnp.exp(-score))
        out_v[pl.ds(b * L, L)] = prob
        return carry

    lax.fori_loop(0, B_PER_W // L, block, 0, unroll=False)

    pltpu.sync_copy(out_v, out_hbm.at[pl.ds(base, B_PER_W)])


@jax.jit
def kernel(user_ids, item_ids, user_table, item_table):
    upacked = _pack_tables(user_table.T)
    ipacked = _pack_tables(item_table.T)
    mesh = plsc.VectorSubcoreMesh(core_axis_name="c", subcore_axis_name="s")
    run = pl.kernel(
        _tt_body,
        out_type=jax.ShapeDtypeStruct((BATCH,), jnp.float32),
        mesh=mesh,
        scratch_types=[
            pltpu.VMEM((B_PER_W,), jnp.int32),
            pltpu.VMEM((B_PER_W,), jnp.int32),
            pltpu.VMEM((L, 2 * EMBED_DIM), jnp.float32),
            pltpu.VMEM((L, 2 * EMBED_DIM), jnp.float32),
            pltpu.VMEM((B_PER_W,), jnp.float32),
            pltpu.SemaphoreType.DMA,
            pltpu.SemaphoreType.DMA,
        ],
    )
    return run(user_ids.astype(jnp.int32), item_ids.astype(jnp.int32),
               upacked, ipacked)
